# baseline (speedup 1.0000x reference)
import functools

import jax
import jax.numpy as jnp
from jax import lax
from jax.experimental import pallas as pl
from jax.experimental.pallas import tpu as pltpu
from jax.experimental.pallas import tpu_sc as plsc

B = 16384
D = 64
NC = 2
NS = 16
NW = NC * NS
BPW = B // NW
CHUNK = 128
NCHUNK = BPW // CHUNK
GROUPS = BPW // 16


def _user_gather_kernel(utab_hbm, uid_hbm, urows_hbm, uidx, ubuf, sem):
    wid = lax.axis_index("s") * NC + lax.axis_index("c")
    pltpu.sync_copy(uid_hbm.at[pl.ds(wid * NCHUNK, NCHUNK)], uidx)
    copies = []
    for j in range(NCHUNK):
        copies.append(pltpu.async_copy(
            utab_hbm.at[uidx.at[j]], ubuf.at[pl.ds(j * CHUNK, CHUNK)], sem))
    for c in copies:
        c.wait()
    pltpu.sync_copy(ubuf, urows_hbm.at[pl.ds(wid * BPW, BPW)])


def _item_compute_kernel(itab_hbm, iid_hbm, urows_hbm, w_hbm, b_hbm, out_hbm,
                         iidx, urows, vrows, wvec, bvec, outv, sem):
    wid = lax.axis_index("s") * NC + lax.axis_index("c")

    pltpu.sync_copy(iid_hbm.at[pl.ds(wid * NCHUNK, NCHUNK)], iidx)
    pltpu.sync_copy(w_hbm, wvec)
    pltpu.sync_copy(b_hbm, bvec)

    copies = []
    for j in range(NCHUNK):
        copies.append(pltpu.async_copy(
            itab_hbm.at[iidx.at[j]], vrows.at[pl.ds(j * CHUNK, CHUNK)], sem))
    copies.append(pltpu.async_copy(
        urows_hbm.at[pl.ds(wid * BPW, BPW)], urows, sem))
    for c in copies:
        c.wait()

    iota = lax.broadcasted_iota(jnp.int32, (16,), 0)
    bias = bvec[...]
    bscal = bias[0]
    wvals = [wvec[pl.ds(j * 16, 16)] for j in range(D // 16)]

    def group_body(g, carry):
        acc = bias
        for r in range(16):
            row = g * 16 + r
            t = (urows[row, pl.ds(0, 16)] * vrows[row, pl.ds(0, 16)]) * wvals[0]
            for j in range(1, D // 16):
                t = t + (urows[row, pl.ds(j * 16, 16)]
                         * vrows[row, pl.ds(j * 16, 16)]) * wvals[j]
            s = jnp.sum(t) + bscal
            acc = jnp.where(iota == r, s, acc)
        outv[pl.ds(g * 16, 16)] = acc
        return carry

    lax.fori_loop(0, GROUPS, group_body, 0)

    pltpu.sync_copy(outv, out_hbm.at[pl.ds(wid * BPW, BPW)])


def kernel(user_id, item_id, user_table, item_table, linear_w, linear_b):
    uid2d = user_id.reshape(NW * NCHUNK, CHUNK).astype(jnp.int32)
    iid2d = item_id.reshape(NW * NCHUNK, CHUNK).astype(jnp.int32)
    w = linear_w.reshape(D)
    b16 = jnp.broadcast_to(linear_b.reshape(()), (16,)).astype(jnp.float32)

    mesh = plsc.VectorSubcoreMesh(core_axis_name="c", subcore_axis_name="s")
    params = pltpu.CompilerParams(
        needs_layout_passes=False, use_tc_tiling_on_sc=False)

    gather_u = functools.partial(
        pl.kernel,
        mesh=mesh,
        out_type=jax.ShapeDtypeStruct((B, D), jnp.float32),
        compiler_params=params,
        scratch_types=[
            pltpu.VMEM((NCHUNK, CHUNK), jnp.int32),
            pltpu.VMEM((BPW, D), jnp.float32),
            pltpu.SemaphoreType.DMA,
        ],
    )(_user_gather_kernel)

    item_compute = functools.partial(
        pl.kernel,
        mesh=mesh,
        out_type=jax.ShapeDtypeStruct((B,), jnp.float32),
        compiler_params=params,
        scratch_types=[
            pltpu.VMEM((NCHUNK, CHUNK), jnp.int32),
            pltpu.VMEM((BPW, D), jnp.float32),
            pltpu.VMEM((BPW, D), jnp.float32),
            pltpu.VMEM((D,), jnp.float32),
            pltpu.VMEM((16,), jnp.float32),
            pltpu.VMEM((BPW,), jnp.float32),
            pltpu.SemaphoreType.DMA,
        ],
    )(_item_compute_kernel)

    urows = gather_u(user_table, uid2d)
    return item_compute(item_table, iid2d, urows, w, b16)


# speedup vs baseline: 1.0021x; 1.0021x over previous
import functools

import jax
import jax.numpy as jnp
from jax import lax
from jax.experimental import pallas as pl
from jax.experimental.pallas import tpu as pltpu
from jax.experimental.pallas import tpu_sc as plsc

B = 16384
D = 64
NC = 2
NS = 16
NW = NC * NS
BPW = B // NW
CHUNK = 128
NCHUNK = BPW // CHUNK
GROUPS = BPW // 16


def _user_gather_kernel(utab_hbm, uid_hbm, urows_hbm, uidx, ubuf, sem):
    wid = lax.axis_index("s") * NC + lax.axis_index("c")
    pltpu.sync_copy(uid_hbm.at[pl.ds(wid * NCHUNK, NCHUNK)], uidx)
    copies = []
    for j in range(NCHUNK):
        copies.append(pltpu.async_copy(
            utab_hbm.at[uidx.at[j]], ubuf.at[pl.ds(j * CHUNK, CHUNK)], sem))
    for c in copies:
        c.wait()
    pltpu.sync_copy(ubuf, urows_hbm.at[pl.ds(wid * BPW, BPW)])


def _item_compute_kernel(itab_hbm, iid_hbm, urows_hbm, w_hbm, b_hbm, out_hbm,
                         iidx, urows, vrows, wvec, bvec, outv, sem):
    wid = lax.axis_index("s") * NC + lax.axis_index("c")

    pltpu.sync_copy(iid_hbm.at[pl.ds(wid * NCHUNK, NCHUNK)], iidx)
    pltpu.sync_copy(w_hbm, wvec)
    pltpu.sync_copy(b_hbm, bvec)

    copies = []
    for j in range(NCHUNK):
        copies.append(pltpu.async_copy(
            itab_hbm.at[iidx.at[j]], vrows.at[pl.ds(j * CHUNK, CHUNK)], sem))
    copies.append(pltpu.async_copy(
        urows_hbm.at[pl.ds(wid * BPW, BPW)], urows, sem))
    for c in copies:
        c.wait()

    iota = lax.broadcasted_iota(jnp.int32, (16,), 0)
    bias = bvec[...]
    bscal = bias[0]
    wvals = [wvec[pl.ds(j * 16, 16)] for j in range(D // 16)]

    def group_body(g, carry):
        acc = bias
        for r in range(16):
            row = g * 16 + r
            t = (urows[row, pl.ds(0, 16)] * vrows[row, pl.ds(0, 16)]) * wvals[0]
            for j in range(1, D // 16):
                t = t + (urows[row, pl.ds(j * 16, 16)]
                         * vrows[row, pl.ds(j * 16, 16)]) * wvals[j]
            s = jnp.sum(t) + bscal
            acc = jnp.where(iota == r, s, acc)
        outv[pl.ds(g * 16, 16)] = acc
        return carry

    lax.fori_loop(0, GROUPS, group_body, 0)

    pltpu.sync_copy(outv, out_hbm.at[pl.ds(wid * BPW, BPW)])


def kernel(user_id, item_id, user_table, item_table, linear_w, linear_b):
    uid2d = user_id.reshape(NW * NCHUNK, CHUNK).astype(jnp.int32)
    iid2d = item_id.reshape(NW * NCHUNK, CHUNK).astype(jnp.int32)
    w = linear_w.reshape(D)
    b16 = jnp.broadcast_to(linear_b.reshape(()), (16,)).astype(jnp.float32)

    mesh = plsc.VectorSubcoreMesh(core_axis_name="c", subcore_axis_name="s")
    params = pltpu.CompilerParams(
        needs_layout_passes=False, use_tc_tiling_on_sc=False,
        skip_device_barrier=True)

    gather_u = functools.partial(
        pl.kernel,
        mesh=mesh,
        out_type=jax.ShapeDtypeStruct((B, D), jnp.float32),
        compiler_params=params,
        scratch_types=[
            pltpu.VMEM((NCHUNK, CHUNK), jnp.int32),
            pltpu.VMEM((BPW, D), jnp.float32),
            pltpu.SemaphoreType.DMA,
        ],
    )(_user_gather_kernel)

    item_compute = functools.partial(
        pl.kernel,
        mesh=mesh,
        out_type=jax.ShapeDtypeStruct((B,), jnp.float32),
        compiler_params=params,
        scratch_types=[
            pltpu.VMEM((NCHUNK, CHUNK), jnp.int32),
            pltpu.VMEM((BPW, D), jnp.float32),
            pltpu.VMEM((BPW, D), jnp.float32),
            pltpu.VMEM((D,), jnp.float32),
            pltpu.VMEM((16,), jnp.float32),
            pltpu.VMEM((BPW,), jnp.float32),
            pltpu.SemaphoreType.DMA,
        ],
    )(_item_compute_kernel)

    urows = gather_u(user_table, uid2d)
    return item_compute(item_table, iid2d, urows, w, b16)
